# Initial kernel scaffold; baseline (speedup 1.0000x reference)
import jax
import jax.numpy as jnp
from jax.experimental import pallas as pl


def kernel(h_movie, h_director, h_actor, edge_index, ns1_movie, ns1_director, ns1_actor, ns2_movie, ns2_director, ns2_actor, W_proj_movie, W_proj_director, W_proj_actor, W_gnn1, b_gnn1, W_gnn2, b_gnn2, W_ctx_movie, W_ctx_director, W_ctx_actor, W_hid, W_out):
    raise NotImplementedError("write your pallas kernel here")


# speedup vs baseline: 5.3117x; 5.3117x over previous
import functools

import jax
import jax.numpy as jnp
from jax import lax
from jax.experimental import pallas as pl
from jax.experimental.pallas import tpu as pltpu
from jax.experimental.pallas import tpu_sc as plsc

N_M = 20000
N_D = 10000
N_A = 20000
N = N_M + N_D + N_A
E = 800000

CHUNK = 128
STREAMS = 4
SUP = CHUNK * STREAMS
N_TILES = 16
SUPS_PER_TILE = 98
E_PAD = SUPS_PER_TILE * SUP * N_TILES
N_CHUNK_ROWS = E_PAD // CHUNK

NROW_PAD = 50048
ROWS_PER_TILE = NROW_PAD // N_TILES
DUMMY_DST = N

S_ALL = 40000
SP = 40960
TAIL_CHUNKS_PER_W = SP // (32 * CHUNK)

_mesh = plsc.VectorSubcoreMesh(core_axis_name="c", subcore_axis_name="s")


@functools.partial(
    pl.kernel,
    out_type=(
        jax.ShapeDtypeStruct((NROW_PAD, 32), jnp.float32),
        jax.ShapeDtypeStruct((NROW_PAD, 32), jnp.float32),
    ),
    mesh=_mesh,
    scratch_types=[
        pltpu.VMEM_SHARED((NROW_PAD, 32), jnp.float32),
        pltpu.VMEM((STREAMS, CHUNK), jnp.int32),
        pltpu.VMEM((STREAMS, CHUNK), jnp.int32),
        pltpu.VMEM((SUP, 32), jnp.float32),
        pltpu.SemaphoreType.DMA,
        pltpu.SemaphoreType.DMA,
    ],
    compiler_params=pltpu.CompilerParams(use_tc_tiling_on_sc=False),
)
def _sc_agg(src2, dst2, h_lo, h_hi, zblk, out_lo, out_hi,
            acc, srcv, dstv, rows, gsem, ssem):
    c = lax.axis_index("c")
    s = lax.axis_index("s")

    pltpu.sync_copy(zblk, acc.at[pl.ds(s * ROWS_PER_TILE, ROWS_PER_TILE)])
    plsc.subcore_barrier()

    def run(h_hbm, out_hbm):
        def body(j, carry):
            sup = s * SUPS_PER_TILE + j
            pltpu.sync_copy(src2.at[pl.ds(sup * STREAMS, STREAMS)], srcv)
            pltpu.sync_copy(dst2.at[pl.ds(sup * STREAMS, STREAMS)], dstv)
            descs = []
            for t in range(STREAMS):
                descs.append(pltpu.async_copy(
                    h_hbm.at[srcv.at[t]],
                    rows.at[pl.ds(t * CHUNK, CHUNK)], gsem))
            for d in descs:
                d.wait()
            descs = []
            for t in range(STREAMS):
                descs.append(pltpu.async_copy(
                    rows.at[pl.ds(t * CHUNK, CHUNK)],
                    acc.at[dstv.at[t]], ssem, add=True))
            for d in descs:
                d.wait()
            return carry

        lax.fori_loop(0, SUPS_PER_TILE, body, 0)
        plsc.subcore_barrier()
        pltpu.sync_copy(acc.at[pl.ds(s * ROWS_PER_TILE, ROWS_PER_TILE)],
                        out_hbm.at[pl.ds(s * ROWS_PER_TILE, ROWS_PER_TILE)])

    @pl.when(c == 0)
    def _():
        run(h_lo, out_lo)

    @pl.when(c == 1)
    def _():
        run(h_hi, out_hi)


@functools.partial(
    pl.kernel,
    out_type=jax.ShapeDtypeStruct((SP, 16), jnp.float32),
    mesh=_mesh,
    scratch_types=[
        pltpu.VMEM((CHUNK,), jnp.int32),
        pltpu.VMEM((CHUNK,), jnp.int32),
        pltpu.VMEM((CHUNK,), jnp.int32),
        pltpu.VMEM((CHUNK, 16), jnp.float32),
        pltpu.VMEM((CHUNK, 16), jnp.float32),
        pltpu.VMEM((CHUNK, 16), jnp.float32),
        pltpu.VMEM((CHUNK, 16), jnp.float32),
        pltpu.SemaphoreType.DMA,
    ],
    compiler_params=pltpu.CompilerParams(use_tc_tiling_on_sc=False),
)
def _sc_tail(a_hbm, b_hbm, c_hbm, im_hbm, id_hbm, ia_hbm, v_hbm,
             imv, idv, iav, ra, rb, rc, vb, sem):
    c = lax.axis_index("c")
    s = lax.axis_index("s")
    w = s * 2 + c

    def body(j, carry):
        base = (w * TAIL_CHUNKS_PER_W + j) * CHUNK
        pltpu.sync_copy(im_hbm.at[pl.ds(base, CHUNK)], imv)
        pltpu.sync_copy(id_hbm.at[pl.ds(base, CHUNK)], idv)
        pltpu.sync_copy(ia_hbm.at[pl.ds(base, CHUNK)], iav)
        d1 = pltpu.async_copy(a_hbm.at[imv], ra, sem)
        d2 = pltpu.async_copy(b_hbm.at[idv], rb, sem)
        d3 = pltpu.async_copy(c_hbm.at[iav], rc, sem)
        d1.wait()
        d2.wait()
        d3.wait()

        def inner(i, carry2):
            vb[i, :] = jnp.maximum(ra[i, :] + rb[i, :] + rc[i, :], 0.0)
            return carry2

        lax.fori_loop(0, CHUNK, inner, 0)
        pltpu.sync_copy(vb, v_hbm.at[pl.ds(base, CHUNK)])
        return carry

    lax.fori_loop(0, TAIL_CHUNKS_PER_W, body, 0)


def _mm(x, w, rblk=1000):
    R, K = x.shape
    O = w.shape[1]

    def kern(x_ref, w_ref, o_ref):
        o_ref[...] = jnp.dot(x_ref[...], w_ref[...],
                             preferred_element_type=jnp.float32)

    return pl.pallas_call(
        kern,
        grid=(R // rblk,),
        in_specs=[pl.BlockSpec((rblk, K), lambda i: (i, 0)),
                  pl.BlockSpec((K, O), lambda i: (0, 0))],
        out_specs=pl.BlockSpec((rblk, O), lambda i: (i, 0)),
        out_shape=jax.ShapeDtypeStruct((R, O), jnp.float32),
    )(x, w)


def _mm_bias_relu(x, w, b, rblk=1000):
    R, K = x.shape
    O = w.shape[1]

    def kern(x_ref, w_ref, b_ref, o_ref):
        t = jnp.dot(x_ref[...], w_ref[...],
                    preferred_element_type=jnp.float32) + b_ref[...]
        o_ref[...] = jnp.maximum(t, 0.0)

    return pl.pallas_call(
        kern,
        grid=(R // rblk,),
        in_specs=[pl.BlockSpec((rblk, K), lambda i: (i, 0)),
                  pl.BlockSpec((K, O), lambda i: (0, 0)),
                  pl.BlockSpec((1, O), lambda i: (0, 0))],
        out_specs=pl.BlockSpec((rblk, O), lambda i: (i, 0)),
        out_shape=jax.ShapeDtypeStruct((R, O), jnp.float32),
    )(x, w, b.reshape(1, O))


def _mm_bias_l2norm(x, w, b, rblk=1000):
    R, K = x.shape
    O = w.shape[1]

    def kern(x_ref, w_ref, b_ref, o_ref):
        t = jnp.dot(x_ref[...], w_ref[...],
                    preferred_element_type=jnp.float32) + b_ref[...]
        n = jnp.sqrt(jnp.sum(t * t, axis=1, keepdims=True))
        o_ref[...] = t / jnp.maximum(n, 1e-12)

    return pl.pallas_call(
        kern,
        grid=(R // rblk,),
        in_specs=[pl.BlockSpec((rblk, K), lambda i: (i, 0)),
                  pl.BlockSpec((K, O), lambda i: (0, 0)),
                  pl.BlockSpec((1, O), lambda i: (0, 0))],
        out_specs=pl.BlockSpec((rblk, O), lambda i: (i, 0)),
        out_shape=jax.ShapeDtypeStruct((R, O), jnp.float32),
    )(x, w, b.reshape(1, O))


def _abc_mm(h, w_ctx_d, w_ctx_a, w_hid, rblk=1000):
    R = h.shape[0]
    nb_m = N_M // rblk
    nb_md = (N_M + N_D) // rblk

    def kern(h_ref, wd_ref, wa_ref, wh_ref, o_ref):
        i = pl.program_id(0)
        wh = wh_ref[...]
        wm = wh[:64, :]
        wd = jnp.dot(wd_ref[...], wh[64:80, :], preferred_element_type=jnp.float32)
        wa = jnp.dot(wa_ref[...], wh[80:96, :], preferred_element_type=jnp.float32)
        w = jnp.where(i < nb_m, wm, jnp.where(i < nb_md, wd, wa))
        o_ref[...] = jnp.dot(h_ref[...], w, preferred_element_type=jnp.float32)

    return pl.pallas_call(
        kern,
        grid=(R // rblk,),
        in_specs=[pl.BlockSpec((rblk, 64), lambda i: (i, 0)),
                  pl.BlockSpec((64, 16), lambda i: (0, 0)),
                  pl.BlockSpec((64, 16), lambda i: (0, 0)),
                  pl.BlockSpec((96, 16), lambda i: (0, 0))],
        out_specs=pl.BlockSpec((rblk, 16), lambda i: (i, 0)),
        out_shape=jax.ShapeDtypeStruct((R, 16), jnp.float32),
    )(h, w_ctx_d, w_ctx_a, w_hid)


def _sigmoid_dot(v, w_out, rblk=2048):
    R = v.shape[0]

    def kern(v_ref, w_ref, o_ref):
        p = jnp.sum(v_ref[...] * w_ref[...], axis=1)
        o_ref[...] = jax.nn.sigmoid(p).reshape(rblk // 128, 128)

    return pl.pallas_call(
        kern,
        grid=(R // rblk,),
        in_specs=[pl.BlockSpec((rblk, 16), lambda i: (i, 0)),
                  pl.BlockSpec((1, 16), lambda i: (0, 0))],
        out_specs=pl.BlockSpec((rblk // 128, 128), lambda i: (i, 0)),
        out_shape=jax.ShapeDtypeStruct((R // 128, 128), jnp.float32),
    )(v, w_out.reshape(1, 16))


def kernel(h_movie, h_director, h_actor, edge_index, ns1_movie, ns1_director,
           ns1_actor, ns2_movie, ns2_director, ns2_actor, W_proj_movie,
           W_proj_director, W_proj_actor, W_gnn1, b_gnn1, W_gnn2, b_gnn2,
           W_ctx_movie, W_ctx_director, W_ctx_actor, W_hid, W_out):
    h0 = jnp.concatenate([
        _mm(h_movie, W_proj_movie),
        _mm(h_director, W_proj_director),
        _mm(h_actor, W_proj_actor),
    ], axis=0)

    npad = E_PAD - E
    src2 = jnp.concatenate(
        [edge_index[0], jnp.zeros((npad,), jnp.int32)]).reshape(N_CHUNK_ROWS, CHUNK)
    dst2 = jnp.concatenate(
        [edge_index[1], jnp.full((npad,), DUMMY_DST, jnp.int32)]).reshape(N_CHUNK_ROWS, CHUNK)
    zblk = jnp.zeros((ROWS_PER_TILE, 32), jnp.float32)

    lo, hi = _sc_agg(src2, dst2, h0[:, :32], h0[:, 32:], zblk)
    agg1 = jnp.concatenate([lo[:N], hi[:N]], axis=1)

    h1 = _mm_bias_relu(agg1, W_gnn1, b_gnn1)

    lo, hi = _sc_agg(src2, dst2, h1[:, :32], h1[:, 32:], zblk)
    agg2 = jnp.concatenate([lo[:N], hi[:N]], axis=1)

    h = _mm_bias_l2norm(agg2, W_gnn2, b_gnn2)

    abc = _abc_mm(h, W_ctx_director, W_ctx_actor, W_hid)
    a_rows = abc[:N_M]
    b_rows = abc[N_M:N_M + N_D]
    c_rows = abc[N_M + N_D:]

    spad = SP - S_ALL
    zpad = jnp.zeros((spad,), jnp.int32)
    im = jnp.concatenate([ns1_movie, ns2_movie, zpad])
    idd = jnp.concatenate([ns1_director, ns2_director, zpad])
    ia = jnp.concatenate([ns1_actor, ns2_actor, zpad])

    v = _sc_tail(a_rows, b_rows, c_rows, im, idd, ia)
    x = _sigmoid_dot(v, W_out).reshape(SP)[:S_ALL]

    out_h = h[4353:8029]
    return (h, x, out_h)


# speedup vs baseline: 5.9410x; 1.1185x over previous
import functools

import jax
import jax.numpy as jnp
from jax import lax
from jax.experimental import pallas as pl
from jax.experimental.pallas import tpu as pltpu
from jax.experimental.pallas import tpu_sc as plsc

N_M = 20000
N_D = 10000
N_A = 20000
N = N_M + N_D + N_A
E = 800000

CHUNK = 128
STREAMS = 3
SUP = CHUNK * STREAMS
N_TILES = 16
SUPS_PER_TILE = 132
E_PAD = SUPS_PER_TILE * SUP * N_TILES
N_CHUNK_ROWS = E_PAD // CHUNK

NROW_PAD = 50016
ROWS_PER_TILE = NROW_PAD // N_TILES
DUMMY_DST = N

S_ALL = 40000
SP = 40960
TAIL_CHUNKS_PER_W = SP // (32 * CHUNK)

_mesh = plsc.VectorSubcoreMesh(core_axis_name="c", subcore_axis_name="s")


@functools.partial(
    pl.kernel,
    out_type=(
        jax.ShapeDtypeStruct((NROW_PAD, 32), jnp.float32),
        jax.ShapeDtypeStruct((NROW_PAD, 32), jnp.float32),
    ),
    mesh=_mesh,
    scratch_types=[
        pltpu.VMEM_SHARED((NROW_PAD, 32), jnp.float32),
        pltpu.VMEM((STREAMS, CHUNK), jnp.int32),
        pltpu.VMEM((STREAMS, CHUNK), jnp.int32),
        pltpu.VMEM((SUP, 32), jnp.float32),
        pltpu.VMEM((STREAMS, CHUNK), jnp.int32),
        pltpu.VMEM((STREAMS, CHUNK), jnp.int32),
        pltpu.VMEM((SUP, 32), jnp.float32),
        pltpu.SemaphoreType.DMA,
        pltpu.SemaphoreType.DMA,
        pltpu.SemaphoreType.DMA,
        pltpu.SemaphoreType.DMA,
    ],
    compiler_params=pltpu.CompilerParams(use_tc_tiling_on_sc=False),
)
def _sc_agg(src2, dst2, h_lo, h_hi, zblk, out_lo, out_hi,
            acc, srcA, dstA, rowsA, srcB, dstB, rowsB,
            gsemA, ssemA, gsemB, ssemB):
    c = lax.axis_index("c")
    s = lax.axis_index("s")

    pltpu.sync_copy(zblk, acc.at[pl.ds(s * ROWS_PER_TILE, ROWS_PER_TILE)])
    plsc.subcore_barrier()

    def run(h_hbm, out_hbm):
        def fill_idx(sup, srcv, dstv):
            pltpu.sync_copy(src2.at[pl.ds(sup * STREAMS, STREAMS)], srcv)
            pltpu.sync_copy(dst2.at[pl.ds(sup * STREAMS, STREAMS)], dstv)

        def fire_gathers(srcv, rows, gsem):
            for t in range(STREAMS):
                pltpu.async_copy(h_hbm.at[srcv.at[t]],
                                 rows.at[pl.ds(t * CHUNK, CHUNK)], gsem)

        def drain_gathers(srcv, rows, gsem):
            for t in range(STREAMS):
                pltpu.make_async_copy(h_hbm.at[srcv.at[t]],
                                      rows.at[pl.ds(t * CHUNK, CHUNK)],
                                      gsem).wait()

        def fire_scatters(dstv, rows, ssem):
            for t in range(STREAMS):
                pltpu.async_copy(rows.at[pl.ds(t * CHUNK, CHUNK)],
                                 acc.at[dstv.at[t]], ssem, add=True)

        def drain_scatters(dstv, rows, ssem):
            for t in range(STREAMS):
                pltpu.make_async_copy(rows.at[pl.ds(t * CHUNK, CHUNK)],
                                      acc.at[dstv.at[t]], ssem).wait()

        base = s * SUPS_PER_TILE
        fill_idx(base, srcA, dstA)
        fire_gathers(srcA, rowsA, gsemA)

        def body(k, carry):
            @pl.when(k > 0)
            def _():
                drain_scatters(dstB, rowsB, ssemB)

            fill_idx(base + 2 * k + 1, srcB, dstB)
            fire_gathers(srcB, rowsB, gsemB)
            drain_gathers(srcA, rowsA, gsemA)
            fire_scatters(dstA, rowsA, ssemA)
            drain_gathers(srcB, rowsB, gsemB)
            drain_scatters(dstA, rowsA, ssemA)

            @pl.when(k < SUPS_PER_TILE // 2 - 1)
            def _():
                fill_idx(base + 2 * k + 2, srcA, dstA)
                fire_gathers(srcA, rowsA, gsemA)

            fire_scatters(dstB, rowsB, ssemB)
            return carry

        lax.fori_loop(0, SUPS_PER_TILE // 2, body, 0)
        drain_scatters(dstB, rowsB, ssemB)

        plsc.subcore_barrier()
        pltpu.sync_copy(acc.at[pl.ds(s * ROWS_PER_TILE, ROWS_PER_TILE)],
                        out_hbm.at[pl.ds(s * ROWS_PER_TILE, ROWS_PER_TILE)])

    @pl.when(c == 0)
    def _():
        run(h_lo, out_lo)

    @pl.when(c == 1)
    def _():
        run(h_hi, out_hi)


@functools.partial(
    pl.kernel,
    out_type=jax.ShapeDtypeStruct((SP, 16), jnp.float32),
    mesh=_mesh,
    scratch_types=[
        pltpu.VMEM((CHUNK,), jnp.int32),
        pltpu.VMEM((CHUNK,), jnp.int32),
        pltpu.VMEM((CHUNK,), jnp.int32),
        pltpu.VMEM((CHUNK, 16), jnp.float32),
        pltpu.VMEM((CHUNK, 16), jnp.float32),
        pltpu.VMEM((CHUNK, 16), jnp.float32),
        pltpu.VMEM((CHUNK, 16), jnp.float32),
        pltpu.SemaphoreType.DMA,
    ],
    compiler_params=pltpu.CompilerParams(use_tc_tiling_on_sc=False),
)
def _sc_tail(abc_hbm, im_hbm, id_hbm, ia_hbm, v_hbm,
             imv, idv, iav, ra, rb, rc, vb, sem):
    c = lax.axis_index("c")
    s = lax.axis_index("s")
    w = s * 2 + c

    def body(j, carry):
        base = (w * TAIL_CHUNKS_PER_W + j) * CHUNK
        pltpu.sync_copy(im_hbm.at[pl.ds(base, CHUNK)], imv)
        pltpu.sync_copy(id_hbm.at[pl.ds(base, CHUNK)], idv)
        pltpu.sync_copy(ia_hbm.at[pl.ds(base, CHUNK)], iav)
        for u in range(CHUNK // 16):
            sl = pl.ds(u * 16, 16)
            idv[sl] = idv[sl] + N_M
            iav[sl] = iav[sl] + (N_M + N_D)
        d1 = pltpu.async_copy(abc_hbm.at[imv], ra, sem)
        d2 = pltpu.async_copy(abc_hbm.at[idv], rb, sem)
        d3 = pltpu.async_copy(abc_hbm.at[iav], rc, sem)
        d1.wait()
        d2.wait()
        d3.wait()

        def inner(i, carry2):
            vb[i, :] = jnp.maximum(ra[i, :] + rb[i, :] + rc[i, :], 0.0)
            return carry2

        lax.fori_loop(0, CHUNK, inner, 0)
        pltpu.sync_copy(vb, v_hbm.at[pl.ds(base, CHUNK)])
        return carry

    lax.fori_loop(0, TAIL_CHUNKS_PER_W, body, 0)


_RB = 2000


def _proj_fused(h_movie, h_director, h_actor, wpm, wpd, wpa, w1):
    nb_m = N_M // _RB
    nb_md = (N_M + N_D) // _RB

    def kern(xm_ref, xd_ref, xa_ref, wpm_ref, wpd_ref, wpa_ref, w1_ref,
             lo_ref, hi_ref):
        i = pl.program_id(0)
        w1 = w1_ref[...]
        ym = jnp.dot(xm_ref[...], jnp.dot(wpm_ref[...], w1,
                                          preferred_element_type=jnp.float32),
                     preferred_element_type=jnp.float32)
        wda = jnp.where(i < nb_md, wpd_ref[...], wpa_ref[...])
        xda = jnp.where(i < nb_md, xd_ref[...], xa_ref[...])
        yda = jnp.dot(xda, jnp.dot(wda, w1, preferred_element_type=jnp.float32),
                      preferred_element_type=jnp.float32)
        y = jnp.where(i < nb_m, ym, yda)
        lo_ref[...] = y[:, :32]
        hi_ref[...] = y[:, 32:]

    return pl.pallas_call(
        kern,
        grid=(N // _RB,),
        in_specs=[
            pl.BlockSpec((_RB, 128), lambda i: (jnp.minimum(i, nb_m - 1), 0)),
            pl.BlockSpec((_RB, 64),
                         lambda i: (jnp.clip(i - nb_m, 0, N_D // _RB - 1), 0)),
            pl.BlockSpec((_RB, 64),
                         lambda i: (jnp.clip(i - nb_md, 0, N_A // _RB - 1), 0)),
            pl.BlockSpec((128, 64), lambda i: (0, 0)),
            pl.BlockSpec((64, 64), lambda i: (0, 0)),
            pl.BlockSpec((64, 64), lambda i: (0, 0)),
            pl.BlockSpec((64, 64), lambda i: (0, 0)),
        ],
        out_specs=[pl.BlockSpec((_RB, 32), lambda i: (i, 0)),
                   pl.BlockSpec((_RB, 32), lambda i: (i, 0))],
        out_shape=(jax.ShapeDtypeStruct((NROW_PAD, 32), jnp.float32),
                   jax.ShapeDtypeStruct((NROW_PAD, 32), jnp.float32)),
    )(h_movie, h_director, h_actor, wpm, wpd, wpa, w1)


def _relu_mm_split(lo, hi, b1, w2):
    def kern(lo_ref, hi_ref, b_ref, w_ref, olo_ref, ohi_ref):
        x = jnp.concatenate([lo_ref[...], hi_ref[...]], axis=1)
        t = jnp.maximum(x + b_ref[...], 0.0)
        y = jnp.dot(t, w_ref[...], preferred_element_type=jnp.float32)
        olo_ref[...] = y[:, :32]
        ohi_ref[...] = y[:, 32:]

    return pl.pallas_call(
        kern,
        grid=(N // _RB,),
        in_specs=[pl.BlockSpec((_RB, 32), lambda i: (i, 0)),
                  pl.BlockSpec((_RB, 32), lambda i: (i, 0)),
                  pl.BlockSpec((1, 64), lambda i: (0, 0)),
                  pl.BlockSpec((64, 64), lambda i: (0, 0))],
        out_specs=[pl.BlockSpec((_RB, 32), lambda i: (i, 0)),
                   pl.BlockSpec((_RB, 32), lambda i: (i, 0))],
        out_shape=(jax.ShapeDtypeStruct((NROW_PAD, 32), jnp.float32),
                   jax.ShapeDtypeStruct((NROW_PAD, 32), jnp.float32)),
    )(lo, hi, b1.reshape(1, 64), w2)


def _norm_abc(lo, hi, b2, w_ctx_d, w_ctx_a, w_hid):
    nb_m = N_M // _RB
    nb_md = (N_M + N_D) // _RB

    def kern(lo_ref, hi_ref, b_ref, wd_ref, wa_ref, wh_ref, h_ref, abc_ref):
        i = pl.program_id(0)
        x = jnp.concatenate([lo_ref[...], hi_ref[...]], axis=1)
        t = x + b_ref[...]
        n = jnp.sqrt(jnp.sum(t * t, axis=1, keepdims=True))
        h = t / jnp.maximum(n, 1e-12)
        h_ref[...] = h
        wh = wh_ref[...]
        wm = wh[:64, :]
        wd = jnp.dot(wd_ref[...], wh[64:80, :], preferred_element_type=jnp.float32)
        wa = jnp.dot(wa_ref[...], wh[80:96, :], preferred_element_type=jnp.float32)
        w = jnp.where(i < nb_m, wm, jnp.where(i < nb_md, wd, wa))
        abc_ref[...] = jnp.dot(h, w, preferred_element_type=jnp.float32)

    return pl.pallas_call(
        kern,
        grid=(N // _RB,),
        in_specs=[pl.BlockSpec((_RB, 32), lambda i: (i, 0)),
                  pl.BlockSpec((_RB, 32), lambda i: (i, 0)),
                  pl.BlockSpec((1, 64), lambda i: (0, 0)),
                  pl.BlockSpec((64, 16), lambda i: (0, 0)),
                  pl.BlockSpec((64, 16), lambda i: (0, 0)),
                  pl.BlockSpec((96, 16), lambda i: (0, 0))],
        out_specs=[pl.BlockSpec((_RB, 64), lambda i: (i, 0)),
                   pl.BlockSpec((_RB, 16), lambda i: (i, 0))],
        out_shape=(jax.ShapeDtypeStruct((N, 64), jnp.float32),
                   jax.ShapeDtypeStruct((N, 16), jnp.float32)),
    )(lo, hi, b2.reshape(1, 64), w_ctx_d, w_ctx_a, w_hid)


def _sigmoid_dot(v, w_out, rblk=2048):
    def kern(v_ref, w_ref, o_ref):
        p = jnp.sum(v_ref[...] * w_ref[...], axis=1)
        o_ref[...] = jax.nn.sigmoid(p).reshape(rblk // 128, 128)

    return pl.pallas_call(
        kern,
        grid=(SP // rblk,),
        in_specs=[pl.BlockSpec((rblk, 16), lambda i: (i, 0)),
                  pl.BlockSpec((1, 16), lambda i: (0, 0))],
        out_specs=pl.BlockSpec((rblk // 128, 128), lambda i: (i, 0)),
        out_shape=jax.ShapeDtypeStruct((SP // 128, 128), jnp.float32),
    )(v, w_out.reshape(1, 16))


def kernel(h_movie, h_director, h_actor, edge_index, ns1_movie, ns1_director,
           ns1_actor, ns2_movie, ns2_director, ns2_actor, W_proj_movie,
           W_proj_director, W_proj_actor, W_gnn1, b_gnn1, W_gnn2, b_gnn2,
           W_ctx_movie, W_ctx_director, W_ctx_actor, W_hid, W_out):
    lo0, hi0 = _proj_fused(h_movie, h_director, h_actor,
                           W_proj_movie, W_proj_director, W_proj_actor, W_gnn1)

    npad = E_PAD - E
    src2 = jnp.concatenate(
        [edge_index[0], jnp.zeros((npad,), jnp.int32)]).reshape(N_CHUNK_ROWS, CHUNK)
    dst2 = jnp.concatenate(
        [edge_index[1], jnp.full((npad,), DUMMY_DST, jnp.int32)]).reshape(N_CHUNK_ROWS, CHUNK)
    zblk = jnp.zeros((ROWS_PER_TILE, 32), jnp.float32)

    a1lo, a1hi = _sc_agg(src2, dst2, lo0, hi0, zblk)
    h1lo, h1hi = _relu_mm_split(a1lo, a1hi, b_gnn1, W_gnn2)
    a2lo, a2hi = _sc_agg(src2, dst2, h1lo, h1hi, zblk)
    h, abc = _norm_abc(a2lo, a2hi, b_gnn2, W_ctx_director, W_ctx_actor, W_hid)

    spad = SP - S_ALL
    zpad = jnp.zeros((spad,), jnp.int32)
    im = jnp.concatenate([ns1_movie, ns2_movie, zpad])
    idd = jnp.concatenate([ns1_director, ns2_director, zpad])
    ia = jnp.concatenate([ns1_actor, ns2_actor, zpad])

    v = _sc_tail(abc, im, idd, ia)
    x = _sigmoid_dot(v, W_out).reshape(SP)[:S_ALL]

    out_h = h[4353:8029]
    return (h, x, out_h)
